# Initial kernel scaffold; baseline (speedup 1.0000x reference)
import jax
import jax.numpy as jnp
from jax.experimental import pallas as pl


def kernel(x, edge_index, W1, b1, W2, b2, Wfc, bfc):
    raise NotImplementedError("write your pallas kernel here")


# speedup vs baseline: 15.9219x; 15.9219x over previous
import jax
import jax.numpy as jnp
from jax import lax
from jax.experimental import pallas as pl
from jax.experimental.pallas import tpu as pltpu
from jax.experimental.pallas import tpu_sc as plsc

N = 50000
E = 800000
IN_CH = 27
HID = 128
H2 = 64

F = 32
NC = 2
NS = 16
LANES = 128
G = 4
RPT = 392
EPAD = RPT * NS * LANES
NACC = 50176
ZROWS = 64
DEG_W = 16
RPT_DEG = RPT // 2
GD = 4
RB = 2000

def _mesh():
    return plsc.VectorSubcoreMesh(
        core_axis_name="c", subcore_axis_name="s",
        num_cores=NC, num_subcores=NS)


def _zero_fill(zbuf, width):
    zv = jnp.zeros((16,), jnp.float32)
    for i in range(ZROWS):
        for k in range(width // 16):
            zbuf[i, k * 16:(k + 1) * 16] = zv


def _zero_acc(acc, zbuf, s):
    tile0 = s * (NACC // NS)
    def zloop(k, carry):
        pltpu.sync_copy(zbuf, acc.at[pl.ds(tile0 + k * ZROWS, ZROWS)])
        return carry
    lax.fori_loop(0, (NACC // NS) // ZROWS, zloop, 0)


def _deg_body(dst2d, d0, d1, acc, zbuf, ones, idx):
    c = lax.axis_index("c")
    s = lax.axis_index("s")
    _zero_fill(zbuf, DEG_W)
    ov = jnp.ones((16,), jnp.float32)
    for i in range(LANES):
        ones[i, 0:16] = ov
    _zero_acc(acc, zbuf, s)
    plsc.subcore_barrier()
    base = c * (NS * RPT_DEG) + s * RPT_DEG
    def body(g, carry):
        pltpu.sync_copy(dst2d.at[pl.ds(base + g * GD, GD)], idx)
        for j in range(GD):
            pltpu.sync_copy(ones, acc.at[idx.at[j]], add=True)
        return carry
    lax.fori_loop(0, RPT_DEG // GD, body, 0)
    plsc.subcore_barrier()
    rows = N // NS
    @pl.when(c == 0)
    def _():
        pltpu.sync_copy(acc.at[pl.ds(s * rows, rows)], d0.at[pl.ds(s * rows, rows)])
    @pl.when(c == 1)
    def _():
        pltpu.sync_copy(acc.at[pl.ds(s * rows, rows)], d1.at[pl.ds(s * rows, rows)])


def _deg_call(dst2d):
    return pl.kernel(
        _deg_body,
        out_type=[jax.ShapeDtypeStruct((N, DEG_W), jnp.float32)] * NC,
        mesh=_mesh(),
        compiler_params=pltpu.CompilerParams(use_tc_tiling_on_sc=False),
        scratch_types=[
            pltpu.VMEM_SHARED((NACC, DEG_W), jnp.float32),
            pltpu.VMEM((ZROWS, DEG_W), jnp.float32),
            pltpu.VMEM((LANES, DEG_W), jnp.float32),
            pltpu.VMEM((GD, LANES), jnp.int32),
        ],
    )(dst2d)


def _agg_pass(chunk_hbm, out_hbm, src2d, dst2d, acc, zbuf, sbuf, sidx, didx,
              sem, s):
    _zero_acc(acc, zbuf, s)
    plsc.subcore_barrier()
    tbase = s * RPT
    def body(g, carry):
        b = tbase + g * G
        pltpu.sync_copy(src2d.at[pl.ds(b, G)], sidx)
        pltpu.sync_copy(dst2d.at[pl.ds(b, G)], didx)
        descs = []
        for j in range(G):
            descs.append(pltpu.async_copy(
                chunk_hbm.at[sidx.at[j]],
                sbuf.at[pl.ds(j * LANES, LANES)], sem))
        for d in descs:
            d.wait()
        for j in range(G):
            pltpu.sync_copy(sbuf.at[pl.ds(j * LANES, LANES)],
                            acc.at[didx.at[j]], add=True)
        return carry
    lax.fori_loop(0, RPT // G, body, 0)
    plsc.subcore_barrier()
    rows = N // NS
    pltpu.sync_copy(acc.at[pl.ds(s * rows, rows)],
                    out_hbm.at[pl.ds(s * rows, rows)])
    plsc.subcore_barrier()


def _make_agg(npc):
    def body(*refs):
        chunks = refs[:NC * npc]
        src2d, dst2d = refs[NC * npc:NC * npc + 2]
        outs = refs[NC * npc + 2:2 * NC * npc + 2]
        acc, zbuf, sbuf, sidx, didx, sem = refs[2 * NC * npc + 2:]
        c = lax.axis_index("c")
        s = lax.axis_index("s")
        _zero_fill(zbuf, F)
        for k in range(npc):
            for cc in range(NC):
                @pl.when(c == cc)
                def _(k=k, cc=cc):
                    _agg_pass(chunks[cc * npc + k], outs[cc * npc + k],
                              src2d, dst2d, acc, zbuf, sbuf, sidx, didx,
                              sem, s)

    def call(chunk_arrays, src2d, dst2d):
        return pl.kernel(
            body,
            out_type=[jax.ShapeDtypeStruct((N, F), jnp.float32)] * (NC * npc),
            mesh=_mesh(),
            compiler_params=pltpu.CompilerParams(use_tc_tiling_on_sc=False),
            scratch_types=[
                pltpu.VMEM_SHARED((NACC, F), jnp.float32),
                pltpu.VMEM((ZROWS, F), jnp.float32),
                pltpu.VMEM((G * LANES, F), jnp.float32),
                pltpu.VMEM((G, LANES), jnp.int32),
                pltpu.VMEM((G, LANES), jnp.int32),
                pltpu.SemaphoreType.DMA,
            ],
        )(*chunk_arrays, src2d, dst2d)
    return call


_agg4 = _make_agg(2)
_agg2 = _make_agg(1)


def _mm1_body(x_ref, w1_ref, d0_ref, d1_ref, hc0, hc1, hc2, hc3, s_ref):
    h = jnp.dot(x_ref[...], w1_ref[...], preferred_element_type=jnp.float32)
    deg = d0_ref[...][:, 0] + d1_ref[...][:, 0] + 1.0
    sv = lax.rsqrt(deg)
    hp = h * sv[:, None]
    hc0[...] = hp[:, 0:32]
    hc1[...] = hp[:, 32:64]
    hc2[...] = hp[:, 64:96]
    hc3[...] = hp[:, 96:128]
    s_ref[...] = sv[:, None]


def _mm1_call(x, W1, d0, d1):
    return pl.pallas_call(
        _mm1_body,
        grid=(N // RB,),
        in_specs=[
            pl.BlockSpec((RB, IN_CH), lambda i: (i, 0)),
            pl.BlockSpec((IN_CH, HID), lambda i: (0, 0)),
            pl.BlockSpec((RB, DEG_W), lambda i: (i, 0)),
            pl.BlockSpec((RB, DEG_W), lambda i: (i, 0)),
        ],
        out_specs=[pl.BlockSpec((RB, F), lambda i: (i, 0))] * 4
        + [pl.BlockSpec((RB, 1), lambda i: (i, 0))],
        out_shape=[jax.ShapeDtypeStruct((N, F), jnp.float32)] * 4
        + [jax.ShapeDtypeStruct((N, 1), jnp.float32)],
    )(x, W1, d0, d1)


def _mid_body(a0, a1, a2, a3, h0, h1, h2, h3, s_ref, b1_ref, w2_ref, g0, g1):
    s = s_ref[...]
    parts = []
    for k, (a, hh) in enumerate(((a0, h0), (a1, h1), (a2, h2), (a3, h3))):
        z = (a[...] + hh[...]) * s + b1_ref[0, k * F:(k + 1) * F]
        parts.append(jnp.maximum(z, 0.0))
    h1full = jnp.concatenate(parts, axis=1)
    t = jnp.dot(h1full, w2_ref[...], preferred_element_type=jnp.float32)
    g = t * s
    g0[...] = g[:, 0:F]
    g1[...] = g[:, F:2 * F]


def _mid_call(aggs, hcs, s, b1, W2):
    return pl.pallas_call(
        _mid_body,
        grid=(N // RB,),
        in_specs=[pl.BlockSpec((RB, F), lambda i: (i, 0))] * 8
        + [
            pl.BlockSpec((RB, 1), lambda i: (i, 0)),
            pl.BlockSpec((1, HID), lambda i: (0, 0)),
            pl.BlockSpec((HID, H2), lambda i: (0, 0)),
        ],
        out_specs=[pl.BlockSpec((RB, F), lambda i: (i, 0))] * 2,
        out_shape=[jax.ShapeDtypeStruct((N, F), jnp.float32)] * 2,
    )(*aggs, *hcs, s, b1, W2)


def _fin_body(e0, e1, g0, g1, s_ref, b2_ref, wfc_ref, bfc_ref, o_ref):
    s = s_ref[...]
    z0 = jnp.maximum((e0[...] + g0[...]) * s + b2_ref[0, 0:F], 0.0)
    z1 = jnp.maximum((e1[...] + g1[...]) * s + b2_ref[0, F:2 * F], 0.0)
    h2 = jnp.concatenate([z0, z1], axis=1)
    y = jnp.sum(h2 * wfc_ref[...], axis=1, keepdims=True) + bfc_ref[0, 0]
    o_ref[...] = jax.nn.sigmoid(y)


def _fin_call(e0, e1, g0, g1, s, b2, wfcT, bfc):
    return pl.pallas_call(
        _fin_body,
        grid=(N // RB,),
        in_specs=[pl.BlockSpec((RB, F), lambda i: (i, 0))] * 4
        + [
            pl.BlockSpec((RB, 1), lambda i: (i, 0)),
            pl.BlockSpec((1, H2), lambda i: (0, 0)),
            pl.BlockSpec((1, H2), lambda i: (0, 0)),
            pl.BlockSpec((1, 1), lambda i: (0, 0)),
        ],
        out_specs=[pl.BlockSpec((RB, 1), lambda i: (i, 0))],
        out_shape=[jax.ShapeDtypeStruct((N, 1), jnp.float32)],
    )(e0, e1, g0, g1, s, b2, wfcT, bfc)[0]


def kernel(x, edge_index, W1, b1, W2, b2, Wfc, bfc):
    src = edge_index[0]
    dst = edge_index[1]
    npad = EPAD - E
    pidx = jnp.arange(npad, dtype=jnp.int32)
    pad_src = pidx % N
    pad_dst = N + pidx % (NACC - N)
    src2d = jnp.concatenate([src, pad_src]).reshape(EPAD // LANES, LANES)
    dst2d = jnp.concatenate([dst, pad_dst]).reshape(EPAD // LANES, LANES)

    d0, d1 = _deg_call(dst2d)
    hc0, hc1, hc2, hc3, s = _mm1_call(x, W1, d0, d1)
    a = _agg4((hc0, hc1, hc2, hc3), src2d, dst2d)
    g0, g1 = _mid_call(a, (hc0, hc1, hc2, hc3), s,
                       b1.reshape(1, HID), W2)
    e0, e1 = _agg2((g0, g1), src2d, dst2d)
    return _fin_call(e0, e1, g0, g1, s, b2.reshape(1, H2),
                     Wfc.reshape(1, H2), bfc.reshape(1, 1))


# speedup vs baseline: 21.7037x; 1.3631x over previous
import jax
import jax.numpy as jnp
from jax import lax
from jax.experimental import pallas as pl
from jax.experimental.pallas import tpu as pltpu
from jax.experimental.pallas import tpu_sc as plsc

N = 50000
E = 800000
IN_CH = 27
HID = 128
H2 = 64

F = 32
NC = 2
NS = 16
LANES = 128
G = 4
RPT = 392
EPAD = RPT * NS * LANES
NACC = 50176
ZROWS = 64
DEG_W = 16
RPT_DEG = RPT // 2
GD = 4
RB = 2000

def _mesh():
    return plsc.VectorSubcoreMesh(
        core_axis_name="c", subcore_axis_name="s",
        num_cores=NC, num_subcores=NS)


def _zero_fill(zbuf, width):
    zv = jnp.zeros((16,), jnp.float32)
    for i in range(ZROWS):
        for k in range(width // 16):
            zbuf[i, k * 16:(k + 1) * 16] = zv


def _zero_acc(acc, zbuf, s):
    tile0 = s * (NACC // NS)
    def zloop(k, carry):
        pltpu.sync_copy(zbuf, acc.at[pl.ds(tile0 + k * ZROWS, ZROWS)])
        return carry
    lax.fori_loop(0, (NACC // NS) // ZROWS, zloop, 0)


def _deg_body(dst2d, d0, d1, acc, zbuf, ones, idx):
    c = lax.axis_index("c")
    s = lax.axis_index("s")
    _zero_fill(zbuf, DEG_W)
    ov = jnp.ones((16,), jnp.float32)
    for i in range(LANES):
        ones[i, 0:16] = ov
    _zero_acc(acc, zbuf, s)
    plsc.subcore_barrier()
    base = c * (NS * RPT_DEG) + s * RPT_DEG
    def body(g, carry):
        pltpu.sync_copy(dst2d.at[pl.ds(base + g * GD, GD)], idx)
        for j in range(GD):
            pltpu.sync_copy(ones, acc.at[idx.at[j]], add=True)
        return carry
    lax.fori_loop(0, RPT_DEG // GD, body, 0)
    plsc.subcore_barrier()
    rows = N // NS
    @pl.when(c == 0)
    def _():
        pltpu.sync_copy(acc.at[pl.ds(s * rows, rows)], d0.at[pl.ds(s * rows, rows)])
    @pl.when(c == 1)
    def _():
        pltpu.sync_copy(acc.at[pl.ds(s * rows, rows)], d1.at[pl.ds(s * rows, rows)])


def _deg_call(dst2d):
    return pl.kernel(
        _deg_body,
        out_type=[jax.ShapeDtypeStruct((N, DEG_W), jnp.float32)] * NC,
        mesh=_mesh(),
        compiler_params=pltpu.CompilerParams(use_tc_tiling_on_sc=False),
        scratch_types=[
            pltpu.VMEM_SHARED((NACC, DEG_W), jnp.float32),
            pltpu.VMEM((ZROWS, DEG_W), jnp.float32),
            pltpu.VMEM((LANES, DEG_W), jnp.float32),
            pltpu.VMEM((GD, LANES), jnp.int32),
        ],
    )(dst2d)


G2 = 2
IB = 28
BB = IB // G2
NBLK = RPT // IB


def _agg_pass(chunk_hbm, out_hbm, src2d, dst2d, acc, zbuf, sbuf, sidx, didx,
              sem0, sem1, s):
    _zero_acc(acc, zbuf, s)
    plsc.subcore_barrier()
    tbase = s * RPT
    sems = (sem0, sem1)

    def blk(b, carry):
        base = tbase + b * IB
        pltpu.sync_copy(src2d.at[pl.ds(base, IB)], sidx)
        pltpu.sync_copy(dst2d.at[pl.ds(base, IB)], didx)
        descs = [None, None]

        def fire(bi):
            h = bi % 2
            ds_ = []
            for j in range(G2):
                ds_.append(pltpu.async_copy(
                    chunk_hbm.at[sidx.at[bi * G2 + j]],
                    sbuf.at[pl.ds((h * G2 + j) * LANES, LANES)],
                    sems[h]))
            descs[h] = ds_

        fire(0)
        fire(1)
        for bi in range(BB):
            h = bi % 2
            for d in descs[h]:
                d.wait()
            for j in range(G2):
                pltpu.sync_copy(sbuf.at[pl.ds((h * G2 + j) * LANES, LANES)],
                                acc.at[didx.at[bi * G2 + j]], add=True)
            if bi + 2 < BB:
                fire(bi + 2)
        return carry

    lax.fori_loop(0, NBLK, blk, 0)
    plsc.subcore_barrier()
    rows = N // NS
    pltpu.sync_copy(acc.at[pl.ds(s * rows, rows)],
                    out_hbm.at[pl.ds(s * rows, rows)])
    plsc.subcore_barrier()


def _make_agg(npc):
    def body(*refs):
        chunks = refs[:NC * npc]
        src2d, dst2d = refs[NC * npc:NC * npc + 2]
        outs = refs[NC * npc + 2:2 * NC * npc + 2]
        acc, zbuf, sbuf, sidx, didx, sem0, sem1 = refs[2 * NC * npc + 2:]
        c = lax.axis_index("c")
        s = lax.axis_index("s")
        _zero_fill(zbuf, F)
        for k in range(npc):
            for cc in range(NC):
                @pl.when(c == cc)
                def _(k=k, cc=cc):
                    _agg_pass(chunks[cc * npc + k], outs[cc * npc + k],
                              src2d, dst2d, acc, zbuf, sbuf, sidx, didx,
                              sem0, sem1, s)

    def call(chunk_arrays, src2d, dst2d):
        return pl.kernel(
            body,
            out_type=[jax.ShapeDtypeStruct((N, F), jnp.float32)] * (NC * npc),
            mesh=_mesh(),
            compiler_params=pltpu.CompilerParams(use_tc_tiling_on_sc=False),
            scratch_types=[
                pltpu.VMEM_SHARED((NACC, F), jnp.float32),
                pltpu.VMEM((ZROWS, F), jnp.float32),
                pltpu.VMEM((2 * G2 * LANES, F), jnp.float32),
                pltpu.VMEM((IB, LANES), jnp.int32),
                pltpu.VMEM((IB, LANES), jnp.int32),
                pltpu.SemaphoreType.DMA,
                pltpu.SemaphoreType.DMA,
            ],
        )(*chunk_arrays, src2d, dst2d)
    return call


_agg4 = _make_agg(2)
_agg2 = _make_agg(1)


def _mm1_body(x_ref, w1_ref, d0_ref, d1_ref, hc0, hc1, hc2, hc3, s_ref):
    h = jnp.dot(x_ref[...], w1_ref[...], preferred_element_type=jnp.float32)
    deg = d0_ref[...][:, 0] + d1_ref[...][:, 0] + 1.0
    sv = lax.rsqrt(deg)
    hp = h * sv[:, None]
    hc0[...] = hp[:, 0:32]
    hc1[...] = hp[:, 32:64]
    hc2[...] = hp[:, 64:96]
    hc3[...] = hp[:, 96:128]
    s_ref[...] = sv[:, None]


def _mm1_call(x, W1, d0, d1):
    return pl.pallas_call(
        _mm1_body,
        grid=(N // RB,),
        in_specs=[
            pl.BlockSpec((RB, IN_CH), lambda i: (i, 0)),
            pl.BlockSpec((IN_CH, HID), lambda i: (0, 0)),
            pl.BlockSpec((RB, DEG_W), lambda i: (i, 0)),
            pl.BlockSpec((RB, DEG_W), lambda i: (i, 0)),
        ],
        out_specs=[pl.BlockSpec((RB, F), lambda i: (i, 0))] * 4
        + [pl.BlockSpec((RB, 1), lambda i: (i, 0))],
        out_shape=[jax.ShapeDtypeStruct((N, F), jnp.float32)] * 4
        + [jax.ShapeDtypeStruct((N, 1), jnp.float32)],
    )(x, W1, d0, d1)


def _mid_body(a0, a1, a2, a3, h0, h1, h2, h3, s_ref, b1_ref, w2_ref, g0, g1):
    s = s_ref[...]
    parts = []
    for k, (a, hh) in enumerate(((a0, h0), (a1, h1), (a2, h2), (a3, h3))):
        z = (a[...] + hh[...]) * s + b1_ref[0, k * F:(k + 1) * F]
        parts.append(jnp.maximum(z, 0.0))
    h1full = jnp.concatenate(parts, axis=1)
    t = jnp.dot(h1full, w2_ref[...], preferred_element_type=jnp.float32)
    g = t * s
    g0[...] = g[:, 0:F]
    g1[...] = g[:, F:2 * F]


def _mid_call(aggs, hcs, s, b1, W2):
    return pl.pallas_call(
        _mid_body,
        grid=(N // RB,),
        in_specs=[pl.BlockSpec((RB, F), lambda i: (i, 0))] * 8
        + [
            pl.BlockSpec((RB, 1), lambda i: (i, 0)),
            pl.BlockSpec((1, HID), lambda i: (0, 0)),
            pl.BlockSpec((HID, H2), lambda i: (0, 0)),
        ],
        out_specs=[pl.BlockSpec((RB, F), lambda i: (i, 0))] * 2,
        out_shape=[jax.ShapeDtypeStruct((N, F), jnp.float32)] * 2,
    )(*aggs, *hcs, s, b1, W2)


def _fin_body(e0, e1, g0, g1, s_ref, b2_ref, wfc_ref, bfc_ref, o_ref):
    s = s_ref[...]
    z0 = jnp.maximum((e0[...] + g0[...]) * s + b2_ref[0, 0:F], 0.0)
    z1 = jnp.maximum((e1[...] + g1[...]) * s + b2_ref[0, F:2 * F], 0.0)
    h2 = jnp.concatenate([z0, z1], axis=1)
    y = jnp.sum(h2 * wfc_ref[...], axis=1, keepdims=True) + bfc_ref[0, 0]
    o_ref[...] = jax.nn.sigmoid(y)


def _fin_call(e0, e1, g0, g1, s, b2, wfcT, bfc):
    return pl.pallas_call(
        _fin_body,
        grid=(N // RB,),
        in_specs=[pl.BlockSpec((RB, F), lambda i: (i, 0))] * 4
        + [
            pl.BlockSpec((RB, 1), lambda i: (i, 0)),
            pl.BlockSpec((1, H2), lambda i: (0, 0)),
            pl.BlockSpec((1, H2), lambda i: (0, 0)),
            pl.BlockSpec((1, 1), lambda i: (0, 0)),
        ],
        out_specs=[pl.BlockSpec((RB, 1), lambda i: (i, 0))],
        out_shape=[jax.ShapeDtypeStruct((N, 1), jnp.float32)],
    )(e0, e1, g0, g1, s, b2, wfcT, bfc)[0]


def kernel(x, edge_index, W1, b1, W2, b2, Wfc, bfc):
    src = edge_index[0]
    dst = edge_index[1]
    npad = EPAD - E
    pidx = jnp.arange(npad, dtype=jnp.int32)
    pad_src = pidx % N
    pad_dst = N + pidx % (NACC - N)
    src2d = jnp.concatenate([src, pad_src]).reshape(EPAD // LANES, LANES)
    dst2d = jnp.concatenate([dst, pad_dst]).reshape(EPAD // LANES, LANES)

    d0, d1 = _deg_call(dst2d)
    hc0, hc1, hc2, hc3, s = _mm1_call(x, W1, d0, d1)
    a = _agg4((hc0, hc1, hc2, hc3), src2d, dst2d)
    g0, g1 = _mid_call(a, (hc0, hc1, hc2, hc3), s,
                       b1.reshape(1, HID), W2)
    e0, e1 = _agg2((g0, g1), src2d, dst2d)
    return _fin_call(e0, e1, g0, g1, s, b2.reshape(1, H2),
                     Wfc.reshape(1, H2), bfc.reshape(1, 1))


# speedup vs baseline: 27.2163x; 1.2540x over previous
import jax
import jax.numpy as jnp
from jax import lax
from jax.experimental import pallas as pl
from jax.experimental.pallas import tpu as pltpu
from jax.experimental.pallas import tpu_sc as plsc

N = 50000
E = 800000
IN_CH = 27
HID = 128
H2 = 64

F = 32
NC = 2
NS = 16
LANES = 128
RPT = 392
EPAD = RPT * NS * LANES
NACC = 50176
ZROWS = 64
DEG_W = 16
RPT_DEG = RPT // 2
GD = 4
RB = 2000

def _mesh():
    return plsc.VectorSubcoreMesh(
        core_axis_name="c", subcore_axis_name="s",
        num_cores=NC, num_subcores=NS)


def _zero_fill(zbuf, width):
    zv = jnp.zeros((16,), jnp.float32)
    for i in range(ZROWS):
        for k in range(width // 16):
            zbuf[i, k * 16:(k + 1) * 16] = zv


def _zero_acc(acc, zbuf, s):
    tile0 = s * (NACC // NS)
    def zloop(k, carry):
        pltpu.sync_copy(zbuf, acc.at[pl.ds(tile0 + k * ZROWS, ZROWS)])
        return carry
    lax.fori_loop(0, (NACC // NS) // ZROWS, zloop, 0)


def _deg_body(dst2d, d0, d1, acc, zbuf, ones, idx):
    c = lax.axis_index("c")
    s = lax.axis_index("s")
    _zero_fill(zbuf, DEG_W)
    ov = jnp.ones((16,), jnp.float32)
    for i in range(LANES):
        ones[i, 0:16] = ov
    _zero_acc(acc, zbuf, s)
    plsc.subcore_barrier()
    base = c * (NS * RPT_DEG) + s * RPT_DEG
    def body(g, carry):
        pltpu.sync_copy(dst2d.at[pl.ds(base + g * GD, GD)], idx)
        for j in range(GD):
            pltpu.sync_copy(ones, acc.at[idx.at[j]], add=True)
        return carry
    lax.fori_loop(0, RPT_DEG // GD, body, 0)
    plsc.subcore_barrier()
    rows = N // NS
    @pl.when(c == 0)
    def _():
        pltpu.sync_copy(acc.at[pl.ds(s * rows, rows)], d0.at[pl.ds(s * rows, rows)])
    @pl.when(c == 1)
    def _():
        pltpu.sync_copy(acc.at[pl.ds(s * rows, rows)], d1.at[pl.ds(s * rows, rows)])


def _deg_call(dst2d):
    return pl.kernel(
        _deg_body,
        out_type=[jax.ShapeDtypeStruct((N, DEG_W), jnp.float32)] * NC,
        mesh=_mesh(),
        compiler_params=pltpu.CompilerParams(use_tc_tiling_on_sc=False),
        scratch_types=[
            pltpu.VMEM_SHARED((NACC, DEG_W), jnp.float32),
            pltpu.VMEM((ZROWS, DEG_W), jnp.float32),
            pltpu.VMEM((LANES, DEG_W), jnp.float32),
            pltpu.VMEM((GD, LANES), jnp.int32),
        ],
    )(dst2d)


G2 = 2
IB = 28
BB = IB // G2
NBLK = RPT // IB


def _agg_pass(view, out_hbm, col0, srcx2d, dst2d, acc, zbuf, sbuf, sidx, didx,
              sem0, sem1, s):
    _zero_acc(acc, zbuf, s)
    plsc.subcore_barrier()
    tbase = s * RPT
    sems = (sem0, sem1)

    def blk(b, carry):
        base = tbase + b * IB
        pltpu.sync_copy(srcx2d.at[pl.ds(base, IB)], sidx)
        pltpu.sync_copy(dst2d.at[pl.ds(base, IB)], didx)
        descs = [None, None]

        def fire(bi):
            h = bi % 2
            ds_ = []
            for j in range(G2):
                ds_.append(pltpu.async_copy(
                    view.at[sidx.at[bi * G2 + j]],
                    sbuf.at[pl.ds((h * G2 + j) * LANES, LANES)],
                    sems[h]))
            descs[h] = ds_

        fire(0)
        fire(1)
        for bi in range(BB):
            h = bi % 2
            for d in descs[h]:
                d.wait()
            for j in range(G2):
                pltpu.sync_copy(sbuf.at[pl.ds((h * G2 + j) * LANES, LANES)],
                                acc.at[didx.at[bi * G2 + j]], add=True)
            if bi + 2 < BB:
                fire(bi + 2)
        return carry

    lax.fori_loop(0, NBLK, blk, 0)
    plsc.subcore_barrier()
    rows = N // NS
    pltpu.sync_copy(acc.at[pl.ds(s * rows, rows)],
                    out_hbm.at[pl.ds(s * rows, rows), pl.ds(col0, F)])
    plsc.subcore_barrier()


def _make_agg(npc, ch, ocols):
    nchunk = NC * npc
    vlen = ch * N - ch + 1

    def body(hx, srcx2d, dst2d, out, acc, zbuf, sbuf, sidx, didx, sem0, sem1):
        c = lax.axis_index("c")
        s = lax.axis_index("s")
        _zero_fill(zbuf, F)
        for cc in range(NC):
            @pl.when(c == cc)
            def _(cc=cc):
                for j in range(npc):
                    k = cc * npc + j
                    _agg_pass(hx.at[pl.ds(k, vlen)], out, F * k,
                              srcx2d, dst2d, acc, zbuf, sbuf, sidx, didx,
                              sem0, sem1, s)

    def call(hx, srcx2d, dst2d):
        return pl.kernel(
            body,
            out_type=[jax.ShapeDtypeStruct((N, ocols), jnp.float32)],
            mesh=_mesh(),
            compiler_params=pltpu.CompilerParams(use_tc_tiling_on_sc=False),
            scratch_types=[
                pltpu.VMEM_SHARED((NACC, F), jnp.float32),
                pltpu.VMEM((ZROWS, F), jnp.float32),
                pltpu.VMEM((2 * G2 * LANES, F), jnp.float32),
                pltpu.VMEM((IB, LANES), jnp.int32),
                pltpu.VMEM((IB, LANES), jnp.int32),
                pltpu.SemaphoreType.DMA,
                pltpu.SemaphoreType.DMA,
            ],
        )(hx, srcx2d, dst2d)[0]
    return call


_agg_l1 = _make_agg(2, 4, HID)
_agg_l2 = _make_agg(1, 2, H2)


def _mm1_body(x_ref, w1_ref, d0_ref, d1_ref, sh_ref, s_ref):
    h = jnp.dot(x_ref[...], w1_ref[...], preferred_element_type=jnp.float32)
    deg = d0_ref[...][:, 0] + d1_ref[...][:, 0] + 1.0
    sv = lax.rsqrt(deg)
    sh_ref[...] = h * sv[:, None]
    s_ref[...] = sv[:, None]


def _mm1_call(x, W1, d0, d1):
    return pl.pallas_call(
        _mm1_body,
        grid=(N // RB,),
        in_specs=[
            pl.BlockSpec((RB, IN_CH), lambda i: (i, 0)),
            pl.BlockSpec((IN_CH, HID), lambda i: (0, 0)),
            pl.BlockSpec((RB, DEG_W), lambda i: (i, 0)),
            pl.BlockSpec((RB, DEG_W), lambda i: (i, 0)),
        ],
        out_specs=[
            pl.BlockSpec((RB, HID), lambda i: (i, 0)),
            pl.BlockSpec((RB, 1), lambda i: (i, 0)),
        ],
        out_shape=[
            jax.ShapeDtypeStruct((N, HID), jnp.float32),
            jax.ShapeDtypeStruct((N, 1), jnp.float32),
        ],
    )(x, W1, d0, d1)


def _mid_body(a_ref, sh_ref, s_ref, b1_ref, w2_ref, g_ref):
    s = s_ref[...]
    z = jnp.maximum((a_ref[...] + sh_ref[...]) * s + b1_ref[...], 0.0)
    g_ref[...] = jnp.dot(z, w2_ref[...],
                         preferred_element_type=jnp.float32) * s


def _mid_call(a, sh, s, b1, W2):
    return pl.pallas_call(
        _mid_body,
        grid=(N // RB,),
        in_specs=[
            pl.BlockSpec((RB, HID), lambda i: (i, 0)),
            pl.BlockSpec((RB, HID), lambda i: (i, 0)),
            pl.BlockSpec((RB, 1), lambda i: (i, 0)),
            pl.BlockSpec((1, HID), lambda i: (0, 0)),
            pl.BlockSpec((HID, H2), lambda i: (0, 0)),
        ],
        out_specs=[pl.BlockSpec((RB, H2), lambda i: (i, 0))],
        out_shape=[jax.ShapeDtypeStruct((N, H2), jnp.float32)],
    )(a, sh, s, b1, W2)[0]


def _fin_body(e_ref, g_ref, s_ref, b2_ref, wfc_ref, bfc_ref, o_ref):
    s = s_ref[...]
    z = jnp.maximum((e_ref[...] + g_ref[...]) * s + b2_ref[...], 0.0)
    y = jnp.sum(z * wfc_ref[...], axis=1, keepdims=True) + bfc_ref[0, 0]
    o_ref[...] = jax.nn.sigmoid(y)


def _fin_call(e, g, s, b2, wfcT, bfc):
    return pl.pallas_call(
        _fin_body,
        grid=(N // RB,),
        in_specs=[
            pl.BlockSpec((RB, H2), lambda i: (i, 0)),
            pl.BlockSpec((RB, H2), lambda i: (i, 0)),
            pl.BlockSpec((RB, 1), lambda i: (i, 0)),
            pl.BlockSpec((1, H2), lambda i: (0, 0)),
            pl.BlockSpec((1, H2), lambda i: (0, 0)),
            pl.BlockSpec((1, 1), lambda i: (0, 0)),
        ],
        out_specs=[pl.BlockSpec((RB, 1), lambda i: (i, 0))],
        out_shape=[jax.ShapeDtypeStruct((N, 1), jnp.float32)],
    )(e, g, s, b2, wfcT, bfc)[0]


def kernel(x, edge_index, W1, b1, W2, b2, Wfc, bfc):
    src = edge_index[0]
    dst = edge_index[1]
    npad = EPAD - E
    pidx = jnp.arange(npad, dtype=jnp.int32)
    pad_src = pidx % N
    pad_dst = N + pidx % (NACC - N)
    srcp = jnp.concatenate([src, pad_src])
    src4_2d = (srcp * 4).reshape(EPAD // LANES, LANES)
    src2_2d = (srcp * 2).reshape(EPAD // LANES, LANES)
    dst2d = jnp.concatenate([dst, pad_dst]).reshape(EPAD // LANES, LANES)

    d0, d1 = _deg_call(dst2d)
    sh, s = _mm1_call(x, W1, d0, d1)
    a = _agg_l1(sh.reshape(4 * N, F), src4_2d, dst2d)
    g = _mid_call(a, sh, s, b1.reshape(1, HID), W2)
    e = _agg_l2(g.reshape(2 * N, F), src2_2d, dst2d)
    return _fin_call(e, g, s, b2.reshape(1, H2),
                     Wfc.reshape(1, H2), bfc.reshape(1, 1))
